# baseline (speedup 1.0000x reference)
import functools

import jax
import jax.numpy as jnp
from jax import lax
from jax.experimental import pallas as pl
from jax.experimental.pallas import tpu as pltpu
from jax.experimental.pallas import tpu_sc as plsc

B = 4096
U = 256
F = 1000
D = 2 * U
NC, NS = 2, 16
NW = NC * NS
BPW = B // NW
CH = 16
NCH = BPW // CH
VL = 16
GPT = D // VL

_sc_mesh = plsc.VectorSubcoreMesh(core_axis_name="c", subcore_axis_name="s")


@functools.partial(
    pl.kernel,
    out_type=(
        jax.ShapeDtypeStruct((B * 4, 128), jnp.float32),
        jax.ShapeDtypeStruct((B,), jnp.float32),
    ),
    mesh=_sc_mesh,
    compiler_params=pltpu.CompilerParams(needs_layout_passes=False),
    scratch_types=[
        pltpu.VMEM((BPW,), jnp.int32),
        pltpu.VMEM((CH, 4, 128), jnp.float32),
        pltpu.VMEM((CH, 4, 128), jnp.float32),
        pltpu.VMEM((CH, 4, 128), jnp.float32),
        pltpu.VMEM((CH, 4, 128), jnp.float32),
        pltpu.VMEM((CH * 4, 128), jnp.float32),
        pltpu.VMEM((CH * 4, 128), jnp.float32),
        pltpu.VMEM((CH * 4, 128), jnp.float32),
        pltpu.VMEM((CH * 4, 128), jnp.float32),
        pltpu.VMEM((F,), jnp.float32),
        pltpu.VMEM((F,), jnp.float32),
        pltpu.VMEM((BPW,), jnp.float32),
        pltpu.VMEM((BPW,), jnp.float32),
        pltpu.SemaphoreType.DMA,
        pltpu.SemaphoreType.DMA,
        pltpu.SemaphoreType.DMA,
        pltpu.SemaphoreType.DMA,
    ],
)
def _sc_sample(mu_hbm, ls_hbm, eps_hbm, ids_hbm, cmu_hbm, cls_hbm, epsc_hbm,
               energies_hbm, conc_hbm,
               ids_v, mu_v0, mu_v1, ls_v0, ls_v1, eps_v0, eps_v1,
               out_v0, out_v1, cmu_v, cls_v, epsc_v, conc_v,
               sem_g0, sem_g1, sem_o0, sem_o1):
    wid = lax.axis_index("s") * NC + lax.axis_index("c")
    base = wid * BPW
    mu_b, ls_b = (mu_v0, mu_v1), (ls_v0, ls_v1)
    eps_b, out_b = (eps_v0, eps_v1), (out_v0, out_v1)
    sem_g, sem_o = (sem_g0, sem_g1), (sem_o0, sem_o1)

    pltpu.sync_copy(ids_hbm.at[pl.ds(base, BPW)], ids_v)

    def in_copies(c, b):
        idx = ids_v.at[pl.ds(c * CH, CH)]
        o = (base + c * CH) * 4
        return (
            pltpu.make_async_copy(mu_hbm.at[idx], mu_b[b], sem_g[b]),
            pltpu.make_async_copy(ls_hbm.at[idx], ls_b[b], sem_g[b]),
            pltpu.make_async_copy(eps_hbm.at[pl.ds(o, CH * 4)], eps_b[b],
                                  sem_g[b]),
        )

    def out_copy(c, b):
        return pltpu.make_async_copy(
            out_b[b], energies_hbm.at[pl.ds((base + c * CH) * 4, CH * 4)],
            sem_o[b])

    for dsc in in_copies(0, 0) + in_copies(1, 1):
        dsc.start()

    pltpu.sync_copy(cmu_hbm, cmu_v)
    pltpu.sync_copy(cls_hbm, cls_v)
    pltpu.sync_copy(epsc_hbm.at[pl.ds(base, BPW)], epsc_v)

    def conc_body(t, _):
        ids16 = ids_v[pl.ds(t * VL, VL)]
        cm = plsc.load_gather(cmu_v, [ids16])
        cs = jnp.exp(plsc.load_gather(cls_v, [ids16]))
        ec = epsc_v[pl.ds(t * VL, VL)]
        conc_v[pl.ds(t * VL, VL)] = jnp.exp(cm + cs * ec)
        return 0

    lax.fori_loop(0, BPW // VL, conc_body, 0)
    pltpu.sync_copy(conc_v, conc_hbm.at[pl.ds(base, BPW)])

    def outer(io, _):
        for b in (0, 1):
            c = io * 2 + b
            for dsc in in_copies(c, b):
                dsc.wait()

            @pl.when(c >= 2)
            def _():
                out_copy(c - 2, b).wait()

            mu_v, ls_v, eps_v, out_v = mu_b[b], ls_b[b], eps_b[b], out_b[b]

            def fma_body(k, _):
                i = k // 4
                r = k % 4
                for g in range(8):
                    col = g * VL
                    mu = mu_v[i, r, pl.ds(col, VL)]
                    sg = jnp.exp(ls_v[i, r, pl.ds(col, VL)])
                    ep = eps_v[k, pl.ds(col, VL)]
                    out_v[k, pl.ds(col, VL)] = mu + sg * ep
                return 0

            lax.fori_loop(0, CH * 4, fma_body, 0)

            @pl.when(c + 2 < NCH)
            def _():
                for dsc in in_copies(c + 2, b):
                    dsc.start()

            out_copy(c, b).start()
        return 0

    lax.fori_loop(0, NCH // 2, outer, 0)
    out_copy(NCH - 2, 0).wait()
    out_copy(NCH - 1, 1).wait()


def kernel(interaction_mu, interaction_log_sigma, conc_mu, conc_log_sigma,
           eps_energy, eps_conc, family_ids):
    mu_rec = (interaction_mu.reshape(2, 128, F, 2)
              .transpose(2, 0, 3, 1).reshape(F, 4, 128))
    ls_rec = (interaction_log_sigma.reshape(2, 128, F, 2)
              .transpose(2, 0, 3, 1).reshape(F, 4, 128))
    eps_rec = (eps_energy.reshape(B, 2, 128, 2)
               .transpose(0, 1, 3, 2).reshape(B * 4, 128))

    out_rec, conc = _sc_sample(mu_rec, ls_rec, eps_rec, family_ids,
                               conc_mu, conc_log_sigma, eps_conc)

    energies = (out_rec.reshape(B, 2, 2, 128)
                .transpose(0, 1, 3, 2).reshape(B, U, 2))
    return energies, conc, family_ids


# speedup vs baseline: 1.0153x; 1.0153x over previous
import functools

import jax
import jax.numpy as jnp
from jax import lax
from jax.experimental import pallas as pl
from jax.experimental.pallas import tpu as pltpu
from jax.experimental.pallas import tpu_sc as plsc

B = 4096
U = 256
F = 1000
D = 2 * U
NC, NS = 2, 16
NW = NC * NS
BPW = B // NW
CH = 16
NCH = BPW // CH
VL = 16
GPT = D // VL

_sc_mesh = plsc.VectorSubcoreMesh(core_axis_name="c", subcore_axis_name="s")


@functools.partial(
    pl.kernel,
    out_type=(
        jax.ShapeDtypeStruct((B * 4, 128), jnp.float32),
        jax.ShapeDtypeStruct((B,), jnp.float32),
        jax.ShapeDtypeStruct((B,), jnp.int32),
    ),
    mesh=_sc_mesh,
    compiler_params=pltpu.CompilerParams(needs_layout_passes=False),
    scratch_types=[
        pltpu.VMEM((BPW,), jnp.int32),
        pltpu.VMEM((CH, 4, 128), jnp.float32),
        pltpu.VMEM((CH, 4, 128), jnp.float32),
        pltpu.VMEM((CH, 4, 128), jnp.float32),
        pltpu.VMEM((CH, 4, 128), jnp.float32),
        pltpu.VMEM((CH * 4, 128), jnp.float32),
        pltpu.VMEM((CH * 4, 128), jnp.float32),
        pltpu.VMEM((CH * 4, 128), jnp.float32),
        pltpu.VMEM((CH * 4, 128), jnp.float32),
        pltpu.VMEM((F,), jnp.float32),
        pltpu.VMEM((F,), jnp.float32),
        pltpu.VMEM((BPW,), jnp.float32),
        pltpu.VMEM((BPW,), jnp.float32),
        pltpu.SemaphoreType.DMA,
        pltpu.SemaphoreType.DMA,
        pltpu.SemaphoreType.DMA,
        pltpu.SemaphoreType.DMA,
    ],
)
def _sc_sample(mu_hbm, ls_hbm, eps_hbm, ids_hbm, cmu_hbm, cls_hbm, epsc_hbm,
               energies_hbm, conc_hbm, ids_out_hbm,
               ids_v, mu_v0, mu_v1, ls_v0, ls_v1, eps_v0, eps_v1,
               out_v0, out_v1, cmu_v, cls_v, epsc_v, conc_v,
               sem_g0, sem_g1, sem_o0, sem_o1):
    wid = lax.axis_index("s") * NC + lax.axis_index("c")
    base = wid * BPW
    mu_b, ls_b = (mu_v0, mu_v1), (ls_v0, ls_v1)
    eps_b, out_b = (eps_v0, eps_v1), (out_v0, out_v1)
    sem_g, sem_o = (sem_g0, sem_g1), (sem_o0, sem_o1)

    pltpu.sync_copy(ids_hbm.at[pl.ds(base, BPW)], ids_v)
    pltpu.sync_copy(ids_v, ids_out_hbm.at[pl.ds(base, BPW)])

    def in_copies(c, b):
        idx = ids_v.at[pl.ds(c * CH, CH)]
        o = (base + c * CH) * 4
        return (
            pltpu.make_async_copy(mu_hbm.at[idx], mu_b[b], sem_g[b]),
            pltpu.make_async_copy(ls_hbm.at[idx], ls_b[b], sem_g[b]),
            pltpu.make_async_copy(eps_hbm.at[pl.ds(o, CH * 4)], eps_b[b],
                                  sem_g[b]),
        )

    def out_copy(c, b):
        return pltpu.make_async_copy(
            out_b[b], energies_hbm.at[pl.ds((base + c * CH) * 4, CH * 4)],
            sem_o[b])

    for dsc in in_copies(0, 0) + in_copies(1, 1):
        dsc.start()

    pltpu.sync_copy(cmu_hbm, cmu_v)
    pltpu.sync_copy(cls_hbm, cls_v)
    pltpu.sync_copy(epsc_hbm.at[pl.ds(base, BPW)], epsc_v)

    def conc_body(t, _):
        ids16 = ids_v[pl.ds(t * VL, VL)]
        cm = plsc.load_gather(cmu_v, [ids16])
        cs = jnp.exp(plsc.load_gather(cls_v, [ids16]))
        ec = epsc_v[pl.ds(t * VL, VL)]
        conc_v[pl.ds(t * VL, VL)] = jnp.exp(cm + cs * ec)
        return 0

    lax.fori_loop(0, BPW // VL, conc_body, 0)
    pltpu.sync_copy(conc_v, conc_hbm.at[pl.ds(base, BPW)])

    def outer(io, _):
        for b in (0, 1):
            c = io * 2 + b
            for dsc in in_copies(c, b):
                dsc.wait()

            @pl.when(c >= 2)
            def _():
                out_copy(c - 2, b).wait()

            mu_v, ls_v, eps_v, out_v = mu_b[b], ls_b[b], eps_b[b], out_b[b]

            def fma_body(k, _):
                i = k // 4
                r = k % 4
                for g in range(8):
                    col = g * VL
                    mu = mu_v[i, r, pl.ds(col, VL)]
                    sg = jnp.exp(ls_v[i, r, pl.ds(col, VL)])
                    ep = eps_v[k, pl.ds(col, VL)]
                    out_v[k, pl.ds(col, VL)] = mu + sg * ep
                return 0

            lax.fori_loop(0, CH * 4, fma_body, 0)

            @pl.when(c + 2 < NCH)
            def _():
                for dsc in in_copies(c + 2, b):
                    dsc.start()

            out_copy(c, b).start()
        return 0

    lax.fori_loop(0, NCH // 2, outer, 0)
    out_copy(NCH - 2, 0).wait()
    out_copy(NCH - 1, 1).wait()


def kernel(interaction_mu, interaction_log_sigma, conc_mu, conc_log_sigma,
           eps_energy, eps_conc, family_ids):
    mu_rec = (interaction_mu.reshape(2, 128, F, 2)
              .transpose(2, 0, 3, 1).reshape(F, 4, 128))
    ls_rec = (interaction_log_sigma.reshape(2, 128, F, 2)
              .transpose(2, 0, 3, 1).reshape(F, 4, 128))
    eps_rec = (eps_energy.reshape(B, 2, 128, 2)
               .transpose(0, 1, 3, 2).reshape(B * 4, 128))

    out_rec, conc, ids_out = _sc_sample(mu_rec, ls_rec, eps_rec, family_ids,
                                        conc_mu, conc_log_sigma, eps_conc)

    energies = (out_rec.reshape(B, 2, 2, 128)
                .transpose(0, 1, 3, 2).reshape(B, U, 2))
    return energies, conc, ids_out


# speedup vs baseline: 1.0221x; 1.0067x over previous
import functools

import jax
import jax.numpy as jnp
from jax import lax
from jax.experimental import pallas as pl
from jax.experimental.pallas import tpu as pltpu
from jax.experimental.pallas import tpu_sc as plsc

B = 4096
U = 256
F = 1000
D = 2 * U
NC, NS = 2, 16
NW = NC * NS
BPW = B // NW
CH = 16
NCH = BPW // CH
VL = 16
GPT = D // VL

_sc_mesh = plsc.VectorSubcoreMesh(core_axis_name="c", subcore_axis_name="s")


@functools.partial(
    pl.kernel,
    out_type=(
        jax.ShapeDtypeStruct((B * 4, 128), jnp.float32),
        jax.ShapeDtypeStruct((B,), jnp.float32),
        jax.ShapeDtypeStruct((B,), jnp.int32),
    ),
    mesh=_sc_mesh,
    compiler_params=pltpu.CompilerParams(needs_layout_passes=False),
    scratch_types=[
        pltpu.VMEM((BPW,), jnp.int32),
        pltpu.VMEM((CH, 4, 128), jnp.float32),
        pltpu.VMEM((CH, 4, 128), jnp.float32),
        pltpu.VMEM((CH, 4, 128), jnp.float32),
        pltpu.VMEM((CH, 4, 128), jnp.float32),
        pltpu.VMEM((CH, 4, 128), jnp.float32),
        pltpu.VMEM((CH, 4, 128), jnp.float32),
        pltpu.VMEM((CH, 4, 128), jnp.float32),
        pltpu.VMEM((CH, 4, 128), jnp.float32),
        pltpu.VMEM((CH * 4, 128), jnp.float32),
        pltpu.VMEM((CH * 4, 128), jnp.float32),
        pltpu.VMEM((CH * 4, 128), jnp.float32),
        pltpu.VMEM((CH * 4, 128), jnp.float32),
        pltpu.VMEM((CH * 4, 128), jnp.float32),
        pltpu.VMEM((CH * 4, 128), jnp.float32),
        pltpu.VMEM((F,), jnp.float32),
        pltpu.VMEM((F,), jnp.float32),
        pltpu.VMEM((BPW,), jnp.float32),
        pltpu.VMEM((BPW,), jnp.float32),
        pltpu.SemaphoreType.DMA,
        pltpu.SemaphoreType.DMA,
        pltpu.SemaphoreType.DMA,
        pltpu.SemaphoreType.DMA,
        pltpu.SemaphoreType.DMA,
        pltpu.SemaphoreType.DMA,
    ],
)
def _sc_sample(mu_hbm, ls_hbm, eps_hbm, ids_hbm, cmu_hbm, cls_hbm, epsc_hbm,
               energies_hbm, conc_hbm, ids_out_hbm,
               ids_v, mu_v0, mu_v1, mu_v2, mu_v3, ls_v0, ls_v1, ls_v2, ls_v3,
               eps_v0, eps_v1, eps_v2, eps_v3,
               out_v0, out_v1, cmu_v, cls_v, epsc_v, conc_v,
               sem_g0, sem_g1, sem_g2, sem_g3, sem_o0, sem_o1):
    wid = lax.axis_index("s") * NC + lax.axis_index("c")
    base = wid * BPW
    mu_b, ls_b = (mu_v0, mu_v1, mu_v2, mu_v3), (ls_v0, ls_v1, ls_v2, ls_v3)
    eps_b, out_b = (eps_v0, eps_v1, eps_v2, eps_v3), (out_v0, out_v1)
    sem_g, sem_o = (sem_g0, sem_g1, sem_g2, sem_g3), (sem_o0, sem_o1)

    pltpu.sync_copy(ids_hbm.at[pl.ds(base, BPW)], ids_v)
    pltpu.sync_copy(ids_v, ids_out_hbm.at[pl.ds(base, BPW)])

    def in_copies(c, b):
        idx = ids_v.at[pl.ds(c * CH, CH)]
        o = (base + c * CH) * 4
        return (
            pltpu.make_async_copy(mu_hbm.at[idx], mu_b[b], sem_g[b]),
            pltpu.make_async_copy(ls_hbm.at[idx], ls_b[b], sem_g[b]),
            pltpu.make_async_copy(eps_hbm.at[pl.ds(o, CH * 4)], eps_b[b],
                                  sem_g[b]),
        )

    def out_copy(c, b):
        return pltpu.make_async_copy(
            out_b[b], energies_hbm.at[pl.ds((base + c * CH) * 4, CH * 4)],
            sem_o[b])

    for dsc in (in_copies(0, 0) + in_copies(1, 1)
                + in_copies(2, 2) + in_copies(3, 3)):
        dsc.start()

    pltpu.sync_copy(cmu_hbm, cmu_v)
    pltpu.sync_copy(cls_hbm, cls_v)
    pltpu.sync_copy(epsc_hbm.at[pl.ds(base, BPW)], epsc_v)

    def conc_body(t, _):
        ids16 = ids_v[pl.ds(t * VL, VL)]
        cm = plsc.load_gather(cmu_v, [ids16])
        cs = jnp.exp(plsc.load_gather(cls_v, [ids16]))
        ec = epsc_v[pl.ds(t * VL, VL)]
        conc_v[pl.ds(t * VL, VL)] = jnp.exp(cm + cs * ec)
        return 0

    lax.fori_loop(0, BPW // VL, conc_body, 0)
    pltpu.sync_copy(conc_v, conc_hbm.at[pl.ds(base, BPW)])

    def outer(io, _):
        for b in (0, 1, 2, 3):
            c = io * 4 + b
            for dsc in in_copies(c, b):
                dsc.wait()

            @pl.when(c >= 2)
            def _():
                out_copy(c - 2, b & 1).wait()

            mu_v, ls_v, eps_v = mu_b[b], ls_b[b], eps_b[b]
            out_v = out_b[b & 1]

            def fma_body(k, _):
                i = k // 4
                r = k % 4
                for g in range(8):
                    col = g * VL
                    mu = mu_v[i, r, pl.ds(col, VL)]
                    sg = jnp.exp(ls_v[i, r, pl.ds(col, VL)])
                    ep = eps_v[k, pl.ds(col, VL)]
                    out_v[k, pl.ds(col, VL)] = mu + sg * ep
                return 0

            lax.fori_loop(0, CH * 4, fma_body, 0)

            @pl.when(c + 4 < NCH)
            def _():
                for dsc in in_copies(c + 4, b):
                    dsc.start()

            out_copy(c, b & 1).start()
        return 0

    lax.fori_loop(0, NCH // 4, outer, 0)
    out_copy(NCH - 2, 0).wait()
    out_copy(NCH - 1, 1).wait()


def kernel(interaction_mu, interaction_log_sigma, conc_mu, conc_log_sigma,
           eps_energy, eps_conc, family_ids):
    mu_rec = (interaction_mu.reshape(2, 128, F, 2)
              .transpose(2, 0, 3, 1).reshape(F, 4, 128))
    ls_rec = (interaction_log_sigma.reshape(2, 128, F, 2)
              .transpose(2, 0, 3, 1).reshape(F, 4, 128))
    eps_rec = (eps_energy.reshape(B, 2, 128, 2)
               .transpose(0, 1, 3, 2).reshape(B * 4, 128))

    out_rec, conc, ids_out = _sc_sample(mu_rec, ls_rec, eps_rec, family_ids,
                                        conc_mu, conc_log_sigma, eps_conc)

    energies = (out_rec.reshape(B, 2, 2, 128)
                .transpose(0, 1, 3, 2).reshape(B, U, 2))
    return energies, conc, ids_out


# speedup vs baseline: 1.0260x; 1.0038x over previous
import functools

import jax
import jax.numpy as jnp
from jax import lax
from jax.experimental import pallas as pl
from jax.experimental.pallas import tpu as pltpu
from jax.experimental.pallas import tpu_sc as plsc

B = 4096
U = 256
F = 1000
D = 2 * U
NC, NS = 2, 16
NW = NC * NS
BPW = B // NW
CH = 16
NCH = BPW // CH
VL = 16

_sc_mesh = plsc.VectorSubcoreMesh(core_axis_name="c", subcore_axis_name="s")


@functools.partial(
    pl.kernel,
    out_type=(
        jax.ShapeDtypeStruct((B * 4, 128), jnp.float32),
        jax.ShapeDtypeStruct((B,), jnp.float32),
        jax.ShapeDtypeStruct((B,), jnp.int32),
    ),
    mesh=_sc_mesh,
    compiler_params=pltpu.CompilerParams(needs_layout_passes=False),
    scratch_types=[
        pltpu.VMEM((BPW,), jnp.int32),
        pltpu.VMEM((CH, 4, 128), jnp.float32),
        pltpu.VMEM((CH, 4, 128), jnp.float32),
        pltpu.VMEM((CH, 4, 128), jnp.float32),
        pltpu.VMEM((CH, 4, 128), jnp.float32),
        pltpu.VMEM((CH, 4, 128), jnp.float32),
        pltpu.VMEM((CH, 4, 128), jnp.float32),
        pltpu.VMEM((CH, 4, 128), jnp.float32),
        pltpu.VMEM((CH, 4, 128), jnp.float32),
        pltpu.VMEM((CH * 4, 128), jnp.float32),
        pltpu.VMEM((CH * 4, 128), jnp.float32),
        pltpu.VMEM((CH * 4, 128), jnp.float32),
        pltpu.VMEM((CH * 4, 128), jnp.float32),
        pltpu.VMEM((CH * 4, 128), jnp.float32),
        pltpu.VMEM((CH * 4, 128), jnp.float32),
        pltpu.VMEM((F,), jnp.float32),
        pltpu.VMEM((F,), jnp.float32),
        pltpu.VMEM((BPW,), jnp.float32),
        pltpu.VMEM((BPW,), jnp.float32),
        pltpu.SemaphoreType.DMA,
        pltpu.SemaphoreType.DMA,
        pltpu.SemaphoreType.DMA,
        pltpu.SemaphoreType.DMA,
        pltpu.SemaphoreType.DMA,
        pltpu.SemaphoreType.DMA,
    ],
)
def _sc_sample(mu_hbm, ls_hbm, eps_hbm, ids_hbm, cmu_hbm, cls_hbm, epsc_hbm,
               energies_hbm, conc_hbm, ids_out_hbm,
               ids_v, mu_v0, mu_v1, mu_v2, mu_v3, ls_v0, ls_v1, ls_v2, ls_v3,
               eps_v0, eps_v1, eps_v2, eps_v3,
               out_v0, out_v1, cmu_v, cls_v, epsc_v, conc_v,
               sem_g0, sem_g1, sem_g2, sem_g3, sem_o0, sem_o1):
    wid = lax.axis_index("s") * NC + lax.axis_index("c")
    base = wid * BPW
    mu_b, ls_b = (mu_v0, mu_v1, mu_v2, mu_v3), (ls_v0, ls_v1, ls_v2, ls_v3)
    eps_b, out_b = (eps_v0, eps_v1, eps_v2, eps_v3), (out_v0, out_v1)
    sem_g, sem_o = (sem_g0, sem_g1, sem_g2, sem_g3), (sem_o0, sem_o1)

    pltpu.sync_copy(ids_hbm.at[pl.ds(base, BPW)], ids_v)
    pltpu.sync_copy(ids_v, ids_out_hbm.at[pl.ds(base, BPW)])

    def in_copies(c, b):
        idx = ids_v.at[pl.ds(c * CH, CH)]
        o = (base + c * CH) * 4
        return (
            pltpu.make_async_copy(mu_hbm.at[idx], mu_b[b], sem_g[b]),
            pltpu.make_async_copy(ls_hbm.at[idx], ls_b[b], sem_g[b]),
            pltpu.make_async_copy(eps_hbm.at[pl.ds(o, CH * 4)], eps_b[b],
                                  sem_g[b]),
        )

    def out_copy(c, b):
        return pltpu.make_async_copy(
            out_b[b], energies_hbm.at[pl.ds((base + c * CH) * 4, CH * 4)],
            sem_o[b])

    for dsc in (in_copies(0, 0) + in_copies(1, 1)
                + in_copies(2, 2) + in_copies(3, 3)):
        dsc.start()

    pltpu.sync_copy(cmu_hbm, cmu_v)
    pltpu.sync_copy(cls_hbm, cls_v)
    pltpu.sync_copy(epsc_hbm.at[pl.ds(base, BPW)], epsc_v)

    def conc_body(t, _):
        ids16 = ids_v[pl.ds(t * VL, VL)]
        cm = plsc.load_gather(cmu_v, [ids16])
        cs = jnp.exp(plsc.load_gather(cls_v, [ids16]))
        ec = epsc_v[pl.ds(t * VL, VL)]
        conc_v[pl.ds(t * VL, VL)] = jnp.exp(cm + cs * ec)
        return 0

    lax.fori_loop(0, BPW // VL, conc_body, 0)
    pltpu.sync_copy(conc_v, conc_hbm.at[pl.ds(base, BPW)])

    def outer(io, _):
        for b in (0, 1, 2, 3):
            c = io * 4 + b
            for dsc in in_copies(c, b):
                dsc.wait()

            @pl.when(c >= 2)
            def _():
                out_copy(c - 2, b & 1).wait()

            mu_v, ls_v, eps_v = mu_b[b], ls_b[b], eps_b[b]
            out_v = out_b[b & 1]

            def fma_body(k, _):
                i = k // 4
                r = k % 4
                for g in range(8):
                    col = g * VL
                    mu = mu_v[i, r, pl.ds(col, VL)]
                    sg = jnp.exp(ls_v[i, r, pl.ds(col, VL)])
                    ep = eps_v[k, pl.ds(col, VL)]
                    out_v[k, pl.ds(col, VL)] = mu + sg * ep
                return 0

            lax.fori_loop(0, CH * 4, fma_body, 0)

            @pl.when(c + 4 < NCH)
            def _():
                for dsc in in_copies(c + 4, b):
                    dsc.start()

            out_copy(c, b & 1).start()
        return 0

    lax.fori_loop(0, NCH // 4, outer, 0)
    out_copy(NCH - 2, 0).wait()
    out_copy(NCH - 1, 1).wait()


def kernel(interaction_mu, interaction_log_sigma, conc_mu, conc_log_sigma,
           eps_energy, eps_conc, family_ids):
    mu_rec = (interaction_mu.reshape(2, 128, F, 2)
              .transpose(2, 0, 3, 1).reshape(F, 4, 128))
    ls_rec = (interaction_log_sigma.reshape(2, 128, F, 2)
              .transpose(2, 0, 3, 1).reshape(F, 4, 128))
    eps_rec = (eps_energy.reshape(B, 2, 128, 2)
               .transpose(0, 1, 3, 2).reshape(B * 4, 128))

    out_rec, conc, ids_out = _sc_sample(mu_rec, ls_rec, eps_rec, family_ids,
                                        conc_mu, conc_log_sigma, eps_conc)

    energies = (out_rec.reshape(B, 2, 2, 128)
                .transpose(0, 1, 3, 2).reshape(B, U, 2))
    return energies, conc, ids_out
